# baseline (speedup 1.0000x reference)
import functools

import jax
import jax.numpy as jnp
from jax import lax
from jax.experimental import pallas as pl
from jax.experimental.pallas import tpu as pltpu
from jax.experimental.pallas import tpu_sc as plsc

_V = 100000
_D = 64
_C = 20
_H = 128

_BV = 4096
_NB = pl.cdiv(_V, _BV)


def _sc_gather(jdx_padded, table_t):
    @functools.partial(
        pl.kernel,
        mesh=plsc.VectorSubcoreMesh(core_axis_name="c", subcore_axis_name="s"),
        out_type=jax.ShapeDtypeStruct((_D, _C * 128), jnp.float32),
        scratch_types=[
            pltpu.VMEM((32,), jnp.int32),
            pltpu.VMEM((_D, 128), jnp.float32),
            pltpu.SemaphoreType.DMA,
        ],
        compiler_params=pltpu.CompilerParams(needs_layout_passes=False),
    )
    def _body(jdx_hbm, table_hbm, out_hbm, jdx_v, buf_v, sem):
        wid = lax.axis_index("s") * 2 + lax.axis_index("c")

        @pl.when(wid < _C)
        def _():
            pltpu.sync_copy(jdx_hbm, jdx_v)
            chunk0 = jdx_v[pl.ds(0, 16)]
            chunk1 = jdx_v[pl.ds(16, 16)]
            chunk = jnp.where(wid < 16, chunk0, chunk1)
            lane = lax.broadcasted_iota(jnp.int32, (16,), 0)
            j = jnp.max(jnp.where(lane == wid % 16, chunk, 0))
            pltpu.async_copy(
                table_hbm.at[:, pl.ds(j * 128, 128)],
                out_hbm.at[:, pl.ds(wid * 128, 128)], sem).wait()

    return _body(jdx_padded, table_t)


def _mlp_body(groups_ref, oh_ref, w1_ref, b1_ref, w2t_ref, b2_ref,
              out_ref, h_s, o_s, m_s, s_s, logz_s):
    i = pl.program_id(0)

    @pl.when(i == 0)
    def _():
        h = b1_ref[...]
        for c in range(_C):
            e_c = jnp.dot(groups_ref[:, c * 128:(c + 1) * 128],
                          oh_ref[:, c:c + 1],
                          preferred_element_type=jnp.float32)
            h = h + lax.dot_general(
                e_c, w1_ref[c * _D:(c + 1) * _D, :],
                (((0,), (0,)), ((), ())),
                preferred_element_type=jnp.float32)
        h_s[...] = jnp.maximum(h, 0.0)
        m_s[0] = -jnp.inf
        s_s[0] = 0.0

    @pl.when(i < _NB)
    def _():
        o_blk = lax.dot_general(
            h_s[...], w2t_ref[...], (((1,), (1,)), ((), ())),
            preferred_element_type=jnp.float32) + b2_ref[...]
        cols = i * _BV + lax.broadcasted_iota(jnp.int32, (1, _BV), 1)
        o_blk = jnp.where(cols < _V, o_blk, -jnp.inf)
        o_s[i] = o_blk

        m_old = m_s[0]
        m_new = jnp.maximum(m_old, jnp.max(o_blk))
        s_s[0] = (s_s[0] * jnp.exp(m_old - m_new)
                  + jnp.sum(jnp.exp(o_blk - m_new)))
        m_s[0] = m_new

        @pl.when(i == _NB - 1)
        def _():
            logz_s[0] = m_s[0] + jnp.log(s_s[0])

    @pl.when(i >= _NB)
    def _():
        out_ref[...] = o_s[i - _NB] - logz_s[0]


def kernel(inputs, emb, W1, b1, W2, b2):
    idx = inputs.astype(jnp.int32)
    jdx = idx // 128
    onehot = (lax.broadcasted_iota(jnp.int32, (128, _C), 0)
              == (idx % 128)[None, :]).astype(jnp.float32)

    jdx_padded = jnp.concatenate(
        [jdx, jnp.zeros((32 - _C,), jnp.int32)])
    groups = _sc_gather(jdx_padded, emb.T)

    log_probs = pl.pallas_call(
        _mlp_body,
        grid=(2 * _NB,),
        in_specs=[
            pl.BlockSpec((_D, _C * 128), lambda i: (0, 0)),
            pl.BlockSpec((128, _C), lambda i: (0, 0)),
            pl.BlockSpec((_C * _D, _H), lambda i: (0, 0)),
            pl.BlockSpec((1, _H), lambda i: (0, 0)),
            pl.BlockSpec((_BV, _H), lambda i: (jnp.minimum(i, _NB - 1), 0)),
            pl.BlockSpec((1, _BV), lambda i: (0, jnp.minimum(i, _NB - 1))),
        ],
        out_specs=pl.BlockSpec(
            (1, _BV), lambda i: (0, jnp.maximum(i - _NB, 0))),
        out_shape=jax.ShapeDtypeStruct((1, _V), jnp.float32),
        scratch_shapes=[
            pltpu.VMEM((1, _H), jnp.float32),
            pltpu.VMEM((_NB, 1, _BV), jnp.float32),
            pltpu.SMEM((1,), jnp.float32),
            pltpu.SMEM((1,), jnp.float32),
            pltpu.SMEM((1,), jnp.float32),
        ],
        compiler_params=pltpu.CompilerParams(
            dimension_semantics=("arbitrary",),
        ),
    )(groups, onehot, W1, b1.reshape(1, _H), W2.T, b2.reshape(1, _V))

    return log_probs


# speedup vs baseline: 1.2971x; 1.2971x over previous
import functools

import jax
import jax.numpy as jnp
from jax import lax
from jax.experimental import pallas as pl
from jax.experimental.pallas import tpu as pltpu
from jax.experimental.pallas import tpu_sc as plsc

_V = 100000
_D = 64
_C = 20
_H = 128

_BV = 4096
_NB = pl.cdiv(_V, _BV)


def _sc_gather(jdx_padded, table_t):
    @functools.partial(
        pl.kernel,
        mesh=plsc.VectorSubcoreMesh(core_axis_name="c", subcore_axis_name="s"),
        out_type=jax.ShapeDtypeStruct((_D, _C * 128), jnp.float32),
        scratch_types=[
            pltpu.VMEM((32,), jnp.int32),
            pltpu.VMEM((_D, 128), jnp.float32),
            pltpu.SemaphoreType.DMA,
        ],
        compiler_params=pltpu.CompilerParams(needs_layout_passes=False),
    )
    def _body(jdx_hbm, table_hbm, out_hbm, jdx_v, buf_v, sem):
        wid = lax.axis_index("s") * 2 + lax.axis_index("c")

        @pl.when(wid < _C)
        def _():
            pltpu.sync_copy(jdx_hbm, jdx_v)
            chunk0 = jdx_v[pl.ds(0, 16)]
            chunk1 = jdx_v[pl.ds(16, 16)]
            chunk = jnp.where(wid < 16, chunk0, chunk1)
            lane = lax.broadcasted_iota(jnp.int32, (16,), 0)
            j = jnp.max(jnp.where(lane == wid % 16, chunk, 0))
            pltpu.async_copy(
                table_hbm.at[:, pl.ds(j * 128, 128)], buf_v, sem).wait()
            pltpu.sync_copy(buf_v, out_hbm.at[:, pl.ds(wid * 128, 128)])

    return _body(jdx_padded, table_t)


def _mlp_body(groups_ref, oh_ref, w1_ref, b1_ref, w2t_ref, b2_ref,
              out_ref, h_s, o_s, m_s, s_s, logz_s):
    i = pl.program_id(0)

    @pl.when(i == 0)
    def _():
        h = b1_ref[...]
        for c in range(_C):
            e_c = jnp.dot(groups_ref[:, c * 128:(c + 1) * 128],
                          oh_ref[:, c:c + 1],
                          preferred_element_type=jnp.float32)
            h = h + lax.dot_general(
                e_c, w1_ref[c * _D:(c + 1) * _D, :],
                (((0,), (0,)), ((), ())),
                preferred_element_type=jnp.float32)
        h_s[...] = jnp.maximum(h, 0.0)
        m_s[0] = -jnp.inf
        s_s[0] = 0.0

    @pl.when(i < _NB)
    def _():
        o_blk = lax.dot_general(
            h_s[...], w2t_ref[...], (((1,), (1,)), ((), ())),
            preferred_element_type=jnp.float32) + b2_ref[...]
        cols = i * _BV + lax.broadcasted_iota(jnp.int32, (1, _BV), 1)
        o_blk = jnp.where(cols < _V, o_blk, -jnp.inf)
        o_s[i] = o_blk

        m_old = m_s[0]
        m_new = jnp.maximum(m_old, jnp.max(o_blk))
        s_s[0] = (s_s[0] * jnp.exp(m_old - m_new)
                  + jnp.sum(jnp.exp(o_blk - m_new)))
        m_s[0] = m_new

        @pl.when(i == _NB - 1)
        def _():
            logz_s[0] = m_s[0] + jnp.log(s_s[0])

    @pl.when(i >= _NB)
    def _():
        out_ref[...] = o_s[i - _NB] - logz_s[0]


def kernel(inputs, emb, W1, b1, W2, b2):
    idx = inputs.astype(jnp.int32)
    jdx = idx // 128
    onehot = (lax.broadcasted_iota(jnp.int32, (128, _C), 0)
              == (idx % 128)[None, :]).astype(jnp.float32)

    jdx_padded = jnp.concatenate(
        [jdx, jnp.zeros((32 - _C,), jnp.int32)])
    groups = _sc_gather(jdx_padded, emb.T)

    log_probs = pl.pallas_call(
        _mlp_body,
        grid=(2 * _NB,),
        in_specs=[
            pl.BlockSpec((_D, _C * 128), lambda i: (0, 0)),
            pl.BlockSpec((128, _C), lambda i: (0, 0)),
            pl.BlockSpec((_C * _D, _H), lambda i: (0, 0)),
            pl.BlockSpec((1, _H), lambda i: (0, 0)),
            pl.BlockSpec((_BV, _H), lambda i: (jnp.minimum(i, _NB - 1), 0)),
            pl.BlockSpec((1, _BV), lambda i: (0, jnp.minimum(i, _NB - 1))),
        ],
        out_specs=pl.BlockSpec(
            (1, _BV), lambda i: (0, jnp.maximum(i - _NB, 0))),
        out_shape=jax.ShapeDtypeStruct((1, _V), jnp.float32),
        scratch_shapes=[
            pltpu.VMEM((1, _H), jnp.float32),
            pltpu.VMEM((_NB, 1, _BV), jnp.float32),
            pltpu.SMEM((1,), jnp.float32),
            pltpu.SMEM((1,), jnp.float32),
            pltpu.SMEM((1,), jnp.float32),
        ],
        compiler_params=pltpu.CompilerParams(
            dimension_semantics=("arbitrary",),
        ),
    )(groups, onehot, W1, b1.reshape(1, _H), W2.T, b2.reshape(1, _V))

    return log_probs


# speedup vs baseline: 1.5903x; 1.2260x over previous
import functools

import jax
import jax.numpy as jnp
from jax import lax
from jax.experimental import pallas as pl
from jax.experimental.pallas import tpu as pltpu
from jax.experimental.pallas import tpu_sc as plsc

_V = 100000
_D = 64
_C = 20
_H = 128

_BV = 8192
_NB = pl.cdiv(_V, _BV)


def _sc_gather(jdx_padded, table_t):
    @functools.partial(
        pl.kernel,
        mesh=plsc.VectorSubcoreMesh(core_axis_name="c", subcore_axis_name="s"),
        out_type=jax.ShapeDtypeStruct((_D, _C * 128), jnp.float32),
        scratch_types=[
            pltpu.VMEM((32,), jnp.int32),
            pltpu.VMEM((_D, 128), jnp.float32),
            pltpu.SemaphoreType.DMA,
        ],
        compiler_params=pltpu.CompilerParams(needs_layout_passes=False),
    )
    def _body(jdx_hbm, table_hbm, out_hbm, jdx_v, buf_v, sem):
        wid = lax.axis_index("s") * 2 + lax.axis_index("c")

        @pl.when(wid < _C)
        def _():
            pltpu.sync_copy(jdx_hbm, jdx_v)
            chunk0 = jdx_v[pl.ds(0, 16)]
            chunk1 = jdx_v[pl.ds(16, 16)]
            chunk = jnp.where(wid < 16, chunk0, chunk1)
            lane = lax.broadcasted_iota(jnp.int32, (16,), 0)
            j = jnp.max(jnp.where(lane == wid % 16, chunk, 0))
            pltpu.async_copy(
                table_hbm.at[:, pl.ds(j * 128, 128)], buf_v, sem).wait()
            pltpu.sync_copy(buf_v, out_hbm.at[:, pl.ds(wid * 128, 128)])

    return _body(jdx_padded, table_t)


def _mlp_body(groups_ref, oh_ref, w1_ref, b1_ref, w2t_ref, b2_ref,
              out_ref, h_s, o_s, m_s, s_s, logz_s):
    i = pl.program_id(0)

    @pl.when(i == 0)
    def _():
        h = b1_ref[...]
        for c in range(_C):
            e_c = jnp.dot(groups_ref[:, c * 128:(c + 1) * 128],
                          oh_ref[:, c:c + 1],
                          preferred_element_type=jnp.float32)
            h = h + lax.dot_general(
                e_c, w1_ref[c * _D:(c + 1) * _D, :],
                (((0,), (0,)), ((), ())),
                preferred_element_type=jnp.float32)
        h_s[...] = jnp.maximum(h, 0.0)
        m_s[0] = -jnp.inf
        s_s[0] = 0.0

    @pl.when(i < _NB)
    def _():
        o_blk = lax.dot_general(
            h_s[...], w2t_ref[...], (((1,), (1,)), ((), ())),
            preferred_element_type=jnp.float32) + b2_ref[...]
        cols = i * _BV + lax.broadcasted_iota(jnp.int32, (1, _BV), 1)
        o_blk = jnp.where(cols < _V, o_blk, -jnp.inf)
        o_s[i] = o_blk

        m_old = m_s[0]
        m_new = jnp.maximum(m_old, jnp.max(o_blk))
        s_s[0] = (s_s[0] * jnp.exp(m_old - m_new)
                  + jnp.sum(jnp.exp(o_blk - m_new)))
        m_s[0] = m_new

        @pl.when(i == _NB - 1)
        def _():
            logz_s[0] = m_s[0] + jnp.log(s_s[0])

    @pl.when(i >= _NB)
    def _():
        out_ref[...] = o_s[i - _NB] - logz_s[0]


def kernel(inputs, emb, W1, b1, W2, b2):
    idx = inputs.astype(jnp.int32)
    jdx = idx // 128
    onehot = (lax.broadcasted_iota(jnp.int32, (128, _C), 0)
              == (idx % 128)[None, :]).astype(jnp.float32)

    jdx_padded = jnp.concatenate(
        [jdx, jnp.zeros((32 - _C,), jnp.int32)])
    groups = _sc_gather(jdx_padded, emb.T)

    log_probs = pl.pallas_call(
        _mlp_body,
        grid=(2 * _NB,),
        in_specs=[
            pl.BlockSpec((_D, _C * 128), lambda i: (0, 0)),
            pl.BlockSpec((128, _C), lambda i: (0, 0)),
            pl.BlockSpec((_C * _D, _H), lambda i: (0, 0)),
            pl.BlockSpec((1, _H), lambda i: (0, 0)),
            pl.BlockSpec((_BV, _H), lambda i: (jnp.minimum(i, _NB - 1), 0)),
            pl.BlockSpec((1, _BV), lambda i: (0, jnp.minimum(i, _NB - 1))),
        ],
        out_specs=pl.BlockSpec(
            (1, _BV), lambda i: (0, jnp.maximum(i - _NB, 0))),
        out_shape=jax.ShapeDtypeStruct((1, _V), jnp.float32),
        scratch_shapes=[
            pltpu.VMEM((1, _H), jnp.float32),
            pltpu.VMEM((_NB, 1, _BV), jnp.float32),
            pltpu.SMEM((1,), jnp.float32),
            pltpu.SMEM((1,), jnp.float32),
            pltpu.SMEM((1,), jnp.float32),
        ],
        compiler_params=pltpu.CompilerParams(
            dimension_semantics=("arbitrary",),
        ),
    )(groups, onehot, W1, b1.reshape(1, _H), W2.T, b2.reshape(1, _V))

    return log_probs


# speedup vs baseline: 1.7429x; 1.0960x over previous
import functools

import jax
import jax.numpy as jnp
from jax import lax
from jax.experimental import pallas as pl
from jax.experimental.pallas import tpu as pltpu
from jax.experimental.pallas import tpu_sc as plsc

_V = 100000
_D = 64
_C = 20
_H = 128

_BV = 16384
_NB = pl.cdiv(_V, _BV)


def _sc_gather(jdx_padded, table_t):
    @functools.partial(
        pl.kernel,
        mesh=plsc.VectorSubcoreMesh(core_axis_name="c", subcore_axis_name="s"),
        out_type=jax.ShapeDtypeStruct((_D, _C * 128), jnp.float32),
        scratch_types=[
            pltpu.VMEM((32,), jnp.int32),
            pltpu.VMEM((_D, 128), jnp.float32),
            pltpu.SemaphoreType.DMA,
        ],
        compiler_params=pltpu.CompilerParams(needs_layout_passes=False),
    )
    def _body(jdx_hbm, table_hbm, out_hbm, jdx_v, buf_v, sem):
        wid = lax.axis_index("s") * 2 + lax.axis_index("c")

        @pl.when(wid < _C)
        def _():
            pltpu.sync_copy(jdx_hbm, jdx_v)
            chunk0 = jdx_v[pl.ds(0, 16)]
            chunk1 = jdx_v[pl.ds(16, 16)]
            chunk = jnp.where(wid < 16, chunk0, chunk1)
            lane = lax.broadcasted_iota(jnp.int32, (16,), 0)
            j = jnp.max(jnp.where(lane == wid % 16, chunk, 0))
            pltpu.async_copy(
                table_hbm.at[:, pl.ds(j * 128, 128)], buf_v, sem).wait()
            pltpu.sync_copy(buf_v, out_hbm.at[:, pl.ds(wid * 128, 128)])

    return _body(jdx_padded, table_t)


def _mlp_body(groups_ref, oh_ref, w1_ref, b1_ref, w2t_ref, b2_ref,
              out_ref, h_s, o_s, m_s, s_s, logz_s):
    i = pl.program_id(0)

    @pl.when(i == 0)
    def _():
        h = b1_ref[...]
        for c in range(_C):
            e_c = jnp.dot(groups_ref[:, c * 128:(c + 1) * 128],
                          oh_ref[:, c:c + 1],
                          preferred_element_type=jnp.float32)
            h = h + lax.dot_general(
                e_c, w1_ref[c * _D:(c + 1) * _D, :],
                (((0,), (0,)), ((), ())),
                preferred_element_type=jnp.float32)
        h_s[...] = jnp.maximum(h, 0.0)
        m_s[0] = -jnp.inf
        s_s[0] = 0.0

    @pl.when(i < _NB)
    def _():
        o_blk = lax.dot_general(
            h_s[...], w2t_ref[...], (((1,), (1,)), ((), ())),
            preferred_element_type=jnp.float32) + b2_ref[...]
        cols = i * _BV + lax.broadcasted_iota(jnp.int32, (1, _BV), 1)
        o_blk = jnp.where(cols < _V, o_blk, -jnp.inf)
        o_s[i] = o_blk

        m_old = m_s[0]
        m_new = jnp.maximum(m_old, jnp.max(o_blk))
        s_s[0] = (s_s[0] * jnp.exp(m_old - m_new)
                  + jnp.sum(jnp.exp(o_blk - m_new)))
        m_s[0] = m_new

        @pl.when(i == _NB - 1)
        def _():
            logz_s[0] = m_s[0] + jnp.log(s_s[0])

    @pl.when(i >= _NB)
    def _():
        out_ref[...] = o_s[i - _NB] - logz_s[0]


def kernel(inputs, emb, W1, b1, W2, b2):
    idx = inputs.astype(jnp.int32)
    jdx = idx // 128
    onehot = (lax.broadcasted_iota(jnp.int32, (128, _C), 0)
              == (idx % 128)[None, :]).astype(jnp.float32)

    jdx_padded = jnp.concatenate(
        [jdx, jnp.zeros((32 - _C,), jnp.int32)])
    groups = _sc_gather(jdx_padded, emb.T)

    log_probs = pl.pallas_call(
        _mlp_body,
        grid=(2 * _NB,),
        in_specs=[
            pl.BlockSpec((_D, _C * 128), lambda i: (0, 0)),
            pl.BlockSpec((128, _C), lambda i: (0, 0)),
            pl.BlockSpec((_C * _D, _H), lambda i: (0, 0)),
            pl.BlockSpec((1, _H), lambda i: (0, 0)),
            pl.BlockSpec((_BV, _H), lambda i: (jnp.minimum(i, _NB - 1), 0)),
            pl.BlockSpec((1, _BV), lambda i: (0, jnp.minimum(i, _NB - 1))),
        ],
        out_specs=pl.BlockSpec(
            (1, _BV), lambda i: (0, jnp.maximum(i - _NB, 0))),
        out_shape=jax.ShapeDtypeStruct((1, _V), jnp.float32),
        scratch_shapes=[
            pltpu.VMEM((1, _H), jnp.float32),
            pltpu.VMEM((_NB, 1, _BV), jnp.float32),
            pltpu.SMEM((1,), jnp.float32),
            pltpu.SMEM((1,), jnp.float32),
            pltpu.SMEM((1,), jnp.float32),
        ],
        compiler_params=pltpu.CompilerParams(
            dimension_semantics=("arbitrary",),
        ),
    )(groups, onehot, W1, b1.reshape(1, _H), W2.T, b2.reshape(1, _V))

    return log_probs


# speedup vs baseline: 1.7638x; 1.0120x over previous
import functools

import jax
import jax.numpy as jnp
from jax import lax
from jax.experimental import pallas as pl
from jax.experimental.pallas import tpu as pltpu
from jax.experimental.pallas import tpu_sc as plsc

_V = 100000
_D = 64
_C = 20
_H = 128

_BV = 32768
_NB = pl.cdiv(_V, _BV)


def _sc_gather(jdx_padded, table_t):
    @functools.partial(
        pl.kernel,
        mesh=plsc.VectorSubcoreMesh(core_axis_name="c", subcore_axis_name="s"),
        out_type=jax.ShapeDtypeStruct((_D, _C * 128), jnp.float32),
        scratch_types=[
            pltpu.VMEM((32,), jnp.int32),
            pltpu.VMEM((_D, 128), jnp.float32),
            pltpu.SemaphoreType.DMA,
        ],
        compiler_params=pltpu.CompilerParams(needs_layout_passes=False),
    )
    def _body(jdx_hbm, table_hbm, out_hbm, jdx_v, buf_v, sem):
        wid = lax.axis_index("s") * 2 + lax.axis_index("c")

        @pl.when(wid < _C)
        def _():
            pltpu.sync_copy(jdx_hbm, jdx_v)
            chunk0 = jdx_v[pl.ds(0, 16)]
            chunk1 = jdx_v[pl.ds(16, 16)]
            chunk = jnp.where(wid < 16, chunk0, chunk1)
            lane = lax.broadcasted_iota(jnp.int32, (16,), 0)
            j = jnp.max(jnp.where(lane == wid % 16, chunk, 0))
            pltpu.async_copy(
                table_hbm.at[:, pl.ds(j * 128, 128)], buf_v, sem).wait()
            pltpu.sync_copy(buf_v, out_hbm.at[:, pl.ds(wid * 128, 128)])

    return _body(jdx_padded, table_t)


def _mlp_body(groups_ref, oh_ref, w1_ref, b1_ref, w2t_ref, b2_ref,
              out_ref, h_s, o_s, m_s, s_s, logz_s):
    i = pl.program_id(0)

    @pl.when(i == 0)
    def _():
        h = b1_ref[...]
        for c in range(_C):
            e_c = jnp.dot(groups_ref[:, c * 128:(c + 1) * 128],
                          oh_ref[:, c:c + 1],
                          preferred_element_type=jnp.float32)
            h = h + lax.dot_general(
                e_c, w1_ref[c * _D:(c + 1) * _D, :],
                (((0,), (0,)), ((), ())),
                preferred_element_type=jnp.float32)
        h_s[...] = jnp.maximum(h, 0.0)
        m_s[0] = -jnp.inf
        s_s[0] = 0.0

    @pl.when(i < _NB)
    def _():
        o_blk = lax.dot_general(
            h_s[...], w2t_ref[...], (((1,), (1,)), ((), ())),
            preferred_element_type=jnp.float32) + b2_ref[...]
        cols = i * _BV + lax.broadcasted_iota(jnp.int32, (1, _BV), 1)
        o_blk = jnp.where(cols < _V, o_blk, -jnp.inf)
        o_s[i] = o_blk

        m_old = m_s[0]
        m_new = jnp.maximum(m_old, jnp.max(o_blk))
        s_s[0] = (s_s[0] * jnp.exp(m_old - m_new)
                  + jnp.sum(jnp.exp(o_blk - m_new)))
        m_s[0] = m_new

        @pl.when(i == _NB - 1)
        def _():
            logz_s[0] = m_s[0] + jnp.log(s_s[0])

    @pl.when(i >= _NB)
    def _():
        out_ref[...] = o_s[i - _NB] - logz_s[0]


def kernel(inputs, emb, W1, b1, W2, b2):
    idx = inputs.astype(jnp.int32)
    jdx = idx // 128
    onehot = (lax.broadcasted_iota(jnp.int32, (128, _C), 0)
              == (idx % 128)[None, :]).astype(jnp.float32)

    jdx_padded = jnp.concatenate(
        [jdx, jnp.zeros((32 - _C,), jnp.int32)])
    groups = _sc_gather(jdx_padded, emb.T)

    log_probs = pl.pallas_call(
        _mlp_body,
        grid=(2 * _NB,),
        in_specs=[
            pl.BlockSpec((_D, _C * 128), lambda i: (0, 0)),
            pl.BlockSpec((128, _C), lambda i: (0, 0)),
            pl.BlockSpec((_C * _D, _H), lambda i: (0, 0)),
            pl.BlockSpec((1, _H), lambda i: (0, 0)),
            pl.BlockSpec((_BV, _H), lambda i: (jnp.minimum(i, _NB - 1), 0)),
            pl.BlockSpec((1, _BV), lambda i: (0, jnp.minimum(i, _NB - 1))),
        ],
        out_specs=pl.BlockSpec(
            (1, _BV), lambda i: (0, jnp.maximum(i - _NB, 0))),
        out_shape=jax.ShapeDtypeStruct((1, _V), jnp.float32),
        scratch_shapes=[
            pltpu.VMEM((1, _H), jnp.float32),
            pltpu.VMEM((_NB, 1, _BV), jnp.float32),
            pltpu.SMEM((1,), jnp.float32),
            pltpu.SMEM((1,), jnp.float32),
            pltpu.SMEM((1,), jnp.float32),
        ],
        compiler_params=pltpu.CompilerParams(
            dimension_semantics=("arbitrary",),
        ),
    )(groups, onehot, W1, b1.reshape(1, _H), W2.T, b2.reshape(1, _V))

    return log_probs
